# baseline (speedup 1.0000x reference)
import functools

import jax
import jax.numpy as jnp
from jax import lax
from jax.experimental import pallas as pl
from jax.experimental.pallas import tpu as pltpu
from jax.experimental.pallas import tpu_sc as plsc

VOCAB = 50257
DIM = 128
B_TOKENS = 4096 * 200
NC, NS = 2, 16
NW = NC * NS
B_PER_W = B_TOKENS // NW
SROWS = 4096
NCHUNK = 13
G = 128
NGROUP = B_PER_W // G
W = 1600
NWIN = B_PER_W // W
NKEY = NCHUNK * 16
KPAD = 224


@functools.partial(
    pl.kernel,
    out_type=jax.ShapeDtypeStruct((B_TOKENS, DIM), jnp.float32),
    mesh=plsc.VectorSubcoreMesh(core_axis_name="c", subcore_axis_name="s"),
    compiler_params=pltpu.CompilerParams(needs_layout_passes=False),
    scratch_types=[
        pltpu.VMEM((W,), jnp.int32),
        pltpu.VMEM((W,), jnp.int32),
        pltpu.VMEM((KPAD,), jnp.int32),
        pltpu.VMEM((KPAD,), jnp.int32),
        pltpu.VMEM((B_PER_W,), jnp.int32),
        pltpu.VMEM((B_PER_W,), jnp.int32),
        pltpu.VMEM((2, G), jnp.int32),
        pltpu.VMEM((G,), jnp.int32),
        pltpu.VMEM((2, G, DIM), jnp.float32),
        pltpu.VMEM_SHARED((SROWS, DIM), jnp.float32),
    ] + [pltpu.SemaphoreType.DMA] * 5,
)
def _gather_kernel(table_hbm, idx_hbm, out_hbm, winbuf0, winbuf1, cur_v,
                   start_v, ridxb, posb, posr, clb, rows_v, spm, *sems):
    winbufs = (winbuf0, winbuf1)
    wsa, wsb, gsem, wsem0, wsem1 = sems
    wsem = (wsem0, wsem1)
    winsem = (wsa, wsb)
    sid = lax.axis_index("s")
    wid = sid * NC + lax.axis_index("c")
    base = wid * B_PER_W
    lane = lax.iota(jnp.int32, 16)
    ones = jnp.ones((16,), jnp.int32)
    zeros = jnp.zeros((16,), jnp.int32)

    def win_desc(w, b):
        return pltpu.make_async_copy(
            idx_hbm.at[pl.ds(base + w * W, W)], winbufs[b], winsem[b])

    def scan(per_vreg):
        win_desc(0, 0).start()
        win_desc(1, 1).start()
        for w in range(NWIN):
            b = w % 2
            win_desc(w, b).wait()

            def body(k, _):
                per_vreg(winbufs[b][pl.ds(k * 16, 16)], w * W + k * 16 + lane)
                return ()

            lax.fori_loop(0, W // 16, body, ())
            if w + 2 < NWIN:
                win_desc(w + 2, b).start()

    for k in range(KPAD // 16):
        cur_v[pl.ds(k * 16, 16)] = zeros

    def hist(v, _):
        key = ((v >> 12) << 4) | lane
        plsc.addupdate_scatter(cur_v, [key], ones)

    with jax.named_scope("ph1_hist"):
        scan(hist)

    total = jnp.int32(0)
    for k in range(KPAD // 16):
        v = cur_v[pl.ds(k * 16, 16)]
        ex = plsc.cumsum(v) - v + total
        start_v[pl.ds(k * 16, 16)] = ex
        cur_v[pl.ds(k * 16, 16)] = ex
        total = total + jnp.sum(v)

    def place(v, gpos):
        cid = v >> 12
        key = (cid << 4) | lane
        off = plsc.load_gather(cur_v, [key])
        plsc.addupdate_scatter(cur_v, [key], ones)
        plsc.store_scatter(ridxb, [off], v & (SROWS - 1))
        plsc.store_scatter(posb, [off], gpos | (cid << 20))

    with jax.named_scope("ph2_place"):
        scan(place)

    s_chunk = [start_v[pl.ds(16 * p, 16)][0] for p in range(NCHUNK)] + [jnp.int32(B_PER_W)]

    def gather_desc(g, b):
        return pltpu.make_async_copy(
            spm.at[ridxb.at[pl.ds(g * G, G)]], rows_v.at[b], gsem)

    def write_desc(b):
        return pltpu.make_async_copy(
            rows_v.at[b], out_hbm.at[posr.at[b]], wsem[b])

    def unpack_pos(g, b):
        for j in range(G // 16):
            pk = posb[pl.ds(g * G + j * 16, 16)]
            posr[b, pl.ds(j * 16, 16)] = base + (pk & 0xFFFFF)

    ph3 = jax.named_scope("ph3_passes"); ph3.__enter__()
    for p in range(NCHUNK):
        plsc.subcore_barrier()
        if p < NCHUNK - 1:
            pltpu.sync_copy(
                table_hbm.at[pl.ds(p * SROWS + sid * (SROWS // 16), SROWS // 16)],
                spm.at[pl.ds(sid * (SROWS // 16), SROWS // 16)])
        else:
            rem = VOCAB - (NCHUNK - 1) * SROWS
            per = 64
            pltpu.sync_copy(
                table_hbm.at[pl.ds((NCHUNK - 1) * SROWS + sid * per, per)],
                spm.at[pl.ds(sid * per, per)])
            @pl.when(sid == 0)
            def _():
                tail = rem - 16 * per
                pltpu.sync_copy(
                    table_hbm.at[pl.ds((NCHUNK - 1) * SROWS + 16 * per, tail)],
                    spm.at[pl.ds(16 * per, tail)])
        plsc.subcore_barrier()

        glo = s_chunk[p] >> 7
        ghi = (s_chunk[p + 1] >> 7) if p < NCHUNK - 1 else jnp.int32(NGROUP)
        n = ghi - glo

        def pair(h, _):
            for b in range(2):
                g = glo + 2 * h + b

                @pl.when((g < ghi) & (h > 0))
                def _():
                    write_desc(b).wait()

                @pl.when(g < ghi)
                def _():
                    unpack_pos(g, b)
                    gather_desc(g, b).start()
                    gather_desc(g, b).wait()
                    write_desc(b).start()
            return ()

        lax.fori_loop(0, (n + 1) >> 1, pair, ())
        for b in range(2):
            @pl.when(n >= b + 1)
            def _():
                write_desc(b).wait()

    ph3.__exit__(None, None, None)
    for p in range(1, NCHUNK):
        gb = s_chunk[p] >> 7

        @pl.when(gb < NGROUP)
        def _():
            for j in range(G // 16):
                rv = ridxb[pl.ds(gb * G + j * 16, 16)]
                pk = posb[pl.ds(gb * G + j * 16, 16)]
                clb[pl.ds(j * 16, 16)] = rv + ((pk >> 20) << 12)
                posr[0, pl.ds(j * 16, 16)] = base + (pk & 0xFFFFF)
            pltpu.make_async_copy(table_hbm.at[clb], rows_v.at[0], gsem).start()
            pltpu.make_async_copy(table_hbm.at[clb], rows_v.at[0], gsem).wait()
            write_desc(0).start()
            write_desc(0).wait()


def kernel(x, table):
    idx = x.reshape(-1).astype(jnp.int32)
    out = _gather_kernel(table, idx)
    return out.reshape(x.shape[0], x.shape[1], DIM)


# speedup vs baseline: 1.0525x; 1.0525x over previous
import functools

import jax
import jax.numpy as jnp
from jax import lax
from jax.experimental import pallas as pl
from jax.experimental.pallas import tpu as pltpu
from jax.experimental.pallas import tpu_sc as plsc

VOCAB = 50257
DIM = 128
B_TOKENS = 4096 * 200
NC, NS = 2, 16
NW = NC * NS
B_PER_W = B_TOKENS // NW
SROWS = 4096
NCHUNK = 13
G = 128
NGROUP = B_PER_W // G
W = 1600
NWIN = B_PER_W // W
NKEY = NCHUNK * 16
KPAD = 224


@functools.partial(
    pl.kernel,
    out_type=jax.ShapeDtypeStruct((B_TOKENS, DIM), jnp.float32),
    mesh=plsc.VectorSubcoreMesh(core_axis_name="c", subcore_axis_name="s"),
    compiler_params=pltpu.CompilerParams(needs_layout_passes=False),
    scratch_types=[
        pltpu.VMEM((W,), jnp.int32),
        pltpu.VMEM((W,), jnp.int32),
        pltpu.VMEM((KPAD,), jnp.int32),
        pltpu.VMEM((KPAD,), jnp.int32),
        pltpu.VMEM((B_PER_W,), jnp.int32),
        pltpu.VMEM((4, G), jnp.int32),
        pltpu.VMEM((4, G), jnp.int32),
        pltpu.VMEM((G,), jnp.int32),
        pltpu.VMEM((4, G, DIM), jnp.float32),
        pltpu.VMEM_SHARED((SROWS, DIM), jnp.float32),
    ] + [pltpu.SemaphoreType.DMA] * 10,
)
def _gather_kernel(table_hbm, idx_hbm, out_hbm, winbuf0, winbuf1, cur_v,
                   start_v, pkb, posr, idxr, clb, rows_v, spm, *sems):
    winbufs = (winbuf0, winbuf1)
    winsem = sems[:2]
    gsem = sems[2:6]
    wsem = sems[6:10]
    sid = lax.axis_index("s")
    wid = sid * NC + lax.axis_index("c")
    base = wid * B_PER_W
    lane = lax.iota(jnp.int32, 16)
    ones = jnp.ones((16,), jnp.int32)
    zeros = jnp.zeros((16,), jnp.int32)

    def win_desc(w, b):
        return pltpu.make_async_copy(
            idx_hbm.at[pl.ds(base + w * W, W)], winbufs[b], winsem[b])

    def scan(per_vreg):
        win_desc(0, 0).start()
        win_desc(1, 1).start()
        for w in range(NWIN):
            b = w % 2
            win_desc(w, b).wait()

            def body(k, _):
                per_vreg(winbufs[b][pl.ds(k * 16, 16)], w * W + k * 16 + lane)
                return ()

            lax.fori_loop(0, W // 16, body, ())
            if w + 2 < NWIN:
                win_desc(w + 2, b).start()

    for k in range(KPAD // 16):
        cur_v[pl.ds(k * 16, 16)] = zeros

    def hist(v, _):
        key = ((v >> 12) << 4) | lane
        plsc.addupdate_scatter(cur_v, [key], ones)

    scan(hist)

    total = jnp.int32(0)
    for k in range(KPAD // 16):
        v = cur_v[pl.ds(k * 16, 16)]
        ex = plsc.cumsum(v) - v + total
        start_v[pl.ds(k * 16, 16)] = ex
        cur_v[pl.ds(k * 16, 16)] = ex
        total = total + jnp.sum(v)

    def place(v, gpos):
        cid = v >> 12
        key = (cid << 4) | lane
        off = plsc.load_gather(cur_v, [key])
        plsc.addupdate_scatter(cur_v, [key], ones)
        pk = (v & (SROWS - 1)) | (gpos << 12) | (cid << 27)
        plsc.store_scatter(pkb, [off], pk)

    scan(place)

    s_chunk = [start_v[pl.ds(16 * p, 16)][0] for p in range(NCHUNK)] + [jnp.int32(B_PER_W)]

    def gather_desc(b):
        return pltpu.make_async_copy(
            spm.at[idxr.at[b]], rows_v.at[b], gsem[b])

    def write_desc(b):
        return pltpu.make_async_copy(
            rows_v.at[b], out_hbm.at[posr.at[b]], wsem[b])

    def unpack(g, b):
        for j in range(G // 16):
            pk = pkb[pl.ds(g * G + j * 16, 16)]
            idxr[b, pl.ds(j * 16, 16)] = pk & (SROWS - 1)
            posr[b, pl.ds(j * 16, 16)] = base + ((pk >> 12) & 0x7FFF)

    for p in range(NCHUNK):
        plsc.subcore_barrier()
        if p < NCHUNK - 1:
            pltpu.sync_copy(
                table_hbm.at[pl.ds(p * SROWS + sid * (SROWS // 16), SROWS // 16)],
                spm.at[pl.ds(sid * (SROWS // 16), SROWS // 16)])
        else:
            rem = VOCAB - (NCHUNK - 1) * SROWS
            per = 64
            pltpu.sync_copy(
                table_hbm.at[pl.ds((NCHUNK - 1) * SROWS + sid * per, per)],
                spm.at[pl.ds(sid * per, per)])
            @pl.when(sid == 0)
            def _():
                tail = rem - 16 * per
                pltpu.sync_copy(
                    table_hbm.at[pl.ds((NCHUNK - 1) * SROWS + 16 * per, tail)],
                    spm.at[pl.ds(16 * per, tail)])
        plsc.subcore_barrier()

        glo = s_chunk[p] >> 7
        ghi = (s_chunk[p + 1] >> 7) if p < NCHUNK - 1 else jnp.int32(NGROUP)
        n = ghi - glo

        for s in range(2):
            @pl.when(glo + s < ghi)
            def _(s=s):
                unpack(glo + s, s)
                gather_desc(s).start()

        def quad(h, _):
            for b in range(4):
                g = glo + 4 * h + b
                gp = g + 2
                bp = (b + 2) % 4
                wcond = (gp < ghi) & (h > 0) if b < 2 else (gp < ghi)

                @pl.when(wcond)
                def _(bp=bp):
                    write_desc(bp).wait()

                @pl.when(gp < ghi)
                def _(gp=gp, bp=bp):
                    unpack(gp, bp)
                    gather_desc(bp).start()

                @pl.when(g < ghi)
                def _(g=g, b=b):
                    gather_desc(b).wait()
                    write_desc(b).start()
            return ()

        lax.fori_loop(0, (n + 3) >> 2, quad, ())
        for b in range(4):
            @pl.when(n >= b + 1)
            def _(b=b):
                write_desc(b).wait()

    for p in range(1, NCHUNK):
        gb = s_chunk[p] >> 7

        @pl.when(gb < NGROUP)
        def _():
            for j in range(G // 16):
                pk = pkb[pl.ds(gb * G + j * 16, 16)]
                clb[pl.ds(j * 16, 16)] = (pk & (SROWS - 1)) | ((pk >> 27) << 12)
                posr[0, pl.ds(j * 16, 16)] = base + ((pk >> 12) & 0x7FFF)
            pltpu.make_async_copy(table_hbm.at[clb], rows_v.at[0], gsem[0]).start()
            pltpu.make_async_copy(table_hbm.at[clb], rows_v.at[0], gsem[0]).wait()
            write_desc(0).start()
            write_desc(0).wait()


def kernel(x, table):
    idx = x.reshape(-1).astype(jnp.int32)
    out = _gather_kernel(table, idx)
    return out.reshape(x.shape[0], x.shape[1], DIM)


# speedup vs baseline: 1.1998x; 1.1400x over previous
import functools

import jax
import jax.numpy as jnp
from jax import lax
from jax.experimental import pallas as pl
from jax.experimental.pallas import tpu as pltpu
from jax.experimental.pallas import tpu_sc as plsc

VOCAB = 50257
DIM = 128
B_TOKENS = 4096 * 200
NC, NS = 2, 16
NW = NC * NS
B_PER_W = B_TOKENS // NW
SROWS = 8192
NCHUNK = 7
CSH = 13
G = 64
NGROUP = B_PER_W // G
W = 1600
NWIN = B_PER_W // W
NKEY = NCHUNK * 16
KPAD = 112


@functools.partial(
    pl.kernel,
    out_type=jax.ShapeDtypeStruct((B_TOKENS, DIM), jnp.float32),
    mesh=plsc.VectorSubcoreMesh(core_axis_name="c", subcore_axis_name="s"),
    compiler_params=pltpu.CompilerParams(needs_layout_passes=False),
    scratch_types=[
        pltpu.VMEM((W,), jnp.int32),
        pltpu.VMEM((W,), jnp.int32),
        pltpu.VMEM((KPAD,), jnp.int32),
        pltpu.VMEM((KPAD,), jnp.int32),
        pltpu.VMEM((B_PER_W,), jnp.int32),
        pltpu.VMEM((4, G), jnp.int32),
        pltpu.VMEM((4, G), jnp.int32),
        pltpu.VMEM((G,), jnp.int32),
        pltpu.VMEM((4, G, DIM), jnp.float32),
        pltpu.VMEM_SHARED((SROWS, DIM), jnp.float32),
    ] + [pltpu.SemaphoreType.DMA] * 10,
)
def _gather_kernel(table_hbm, idx_hbm, out_hbm, winbuf0, winbuf1, cur_v,
                   start_v, pkb, posr, idxr, clb, rows_v, spm, *sems):
    winbufs = (winbuf0, winbuf1)
    winsem = sems[:2]
    gsem = sems[2:6]
    wsem = sems[6:10]
    sid = lax.axis_index("s")
    wid = sid * NC + lax.axis_index("c")
    base = wid * B_PER_W
    lane = lax.iota(jnp.int32, 16)
    ones = jnp.ones((16,), jnp.int32)
    zeros = jnp.zeros((16,), jnp.int32)

    def win_desc(w, b):
        return pltpu.make_async_copy(
            idx_hbm.at[pl.ds(base + w * W, W)], winbufs[b], winsem[b])

    def scan(per_vreg):
        win_desc(0, 0).start()
        win_desc(1, 1).start()
        for w in range(NWIN):
            b = w % 2
            win_desc(w, b).wait()

            def body(k, _):
                per_vreg(winbufs[b][pl.ds(k * 16, 16)], w * W + k * 16 + lane)
                return ()

            lax.fori_loop(0, W // 16, body, ())
            if w + 2 < NWIN:
                win_desc(w + 2, b).start()

    for k in range(KPAD // 16):
        cur_v[pl.ds(k * 16, 16)] = zeros

    def hist(v, _):
        key = ((v >> CSH) << 4) | lane
        plsc.addupdate_scatter(cur_v, [key], ones)

    scan(hist)

    total = jnp.int32(0)
    for k in range(KPAD // 16):
        v = cur_v[pl.ds(k * 16, 16)]
        ex = plsc.cumsum(v) - v + total
        start_v[pl.ds(k * 16, 16)] = ex
        cur_v[pl.ds(k * 16, 16)] = ex
        total = total + jnp.sum(v)

    def place(v, gpos):
        cid = v >> CSH
        key = (cid << 4) | lane
        off = plsc.load_gather(cur_v, [key])
        plsc.addupdate_scatter(cur_v, [key], ones)
        pk = (v & (SROWS - 1)) | (gpos << CSH) | (cid << (CSH + 15))
        plsc.store_scatter(pkb, [off], pk)

    scan(place)

    s_chunk = [start_v[pl.ds(16 * p, 16)][0] for p in range(NCHUNK)] + [jnp.int32(B_PER_W)]

    def gather_desc(b):
        return pltpu.make_async_copy(
            spm.at[idxr.at[b]], rows_v.at[b], gsem[b])

    def write_desc(b):
        return pltpu.make_async_copy(
            rows_v.at[b], out_hbm.at[posr.at[b]], wsem[b])

    def unpack(g, b):
        for j in range(G // 16):
            pk = pkb[pl.ds(g * G + j * 16, 16)]
            idxr[b, pl.ds(j * 16, 16)] = pk & (SROWS - 1)
            posr[b, pl.ds(j * 16, 16)] = base + ((pk >> CSH) & 0x7FFF)

    for p in range(NCHUNK):
        plsc.subcore_barrier()
        if p < NCHUNK - 1:
            pltpu.sync_copy(
                table_hbm.at[pl.ds(p * SROWS + sid * (SROWS // 16), SROWS // 16)],
                spm.at[pl.ds(sid * (SROWS // 16), SROWS // 16)])
        else:
            rem = VOCAB - (NCHUNK - 1) * SROWS
            per = 64
            pltpu.sync_copy(
                table_hbm.at[pl.ds((NCHUNK - 1) * SROWS + sid * per, per)],
                spm.at[pl.ds(sid * per, per)])
            @pl.when(sid == 0)
            def _():
                tail = rem - 16 * per
                pltpu.sync_copy(
                    table_hbm.at[pl.ds((NCHUNK - 1) * SROWS + 16 * per, tail)],
                    spm.at[pl.ds(16 * per, tail)])
        plsc.subcore_barrier()

        glo = s_chunk[p] >> 6
        ghi = (s_chunk[p + 1] >> 6) if p < NCHUNK - 1 else jnp.int32(NGROUP)
        n = ghi - glo

        for s in range(2):
            @pl.when(glo + s < ghi)
            def _(s=s):
                unpack(glo + s, s)
                gather_desc(s).start()

        def quad(h, _):
            for b in range(4):
                g = glo + 4 * h + b
                gp = g + 2
                bp = (b + 2) % 4
                wcond = (gp < ghi) & (h > 0) if b < 2 else (gp < ghi)

                @pl.when(wcond)
                def _(bp=bp):
                    write_desc(bp).wait()

                @pl.when(gp < ghi)
                def _(gp=gp, bp=bp):
                    unpack(gp, bp)
                    gather_desc(bp).start()

                @pl.when(g < ghi)
                def _(g=g, b=b):
                    gather_desc(b).wait()
                    write_desc(b).start()
            return ()

        lax.fori_loop(0, (n + 3) >> 2, quad, ())
        for b in range(4):
            @pl.when(n >= b + 1)
            def _(b=b):
                write_desc(b).wait()

    for p in range(1, NCHUNK):
        gb = s_chunk[p] >> 6

        @pl.when(gb < NGROUP)
        def _():
            for j in range(G // 16):
                pk = pkb[pl.ds(gb * G + j * 16, 16)]
                clb[pl.ds(j * 16, 16)] = (pk & (SROWS - 1)) | ((pk >> (CSH + 15)) << CSH)
                posr[0, pl.ds(j * 16, 16)] = base + ((pk >> CSH) & 0x7FFF)
            pltpu.make_async_copy(table_hbm.at[clb], rows_v.at[0], gsem[0]).start()
            pltpu.make_async_copy(table_hbm.at[clb], rows_v.at[0], gsem[0]).wait()
            write_desc(0).start()
            write_desc(0).wait()


def kernel(x, table):
    idx = x.reshape(-1).astype(jnp.int32)
    out = _gather_kernel(table, idx)
    return out.reshape(x.shape[0], x.shape[1], DIM)


# speedup vs baseline: 1.2027x; 1.0024x over previous
import functools

import jax
import jax.numpy as jnp
from jax import lax
from jax.experimental import pallas as pl
from jax.experimental.pallas import tpu as pltpu
from jax.experimental.pallas import tpu_sc as plsc

VOCAB = 50257
DIM = 128
B_TOKENS = 4096 * 200
NC, NS = 2, 16
NW = NC * NS
B_PER_W = B_TOKENS // NW
SROWS = 8192
NCHUNK = 7
CSH = 13
G = 64
NGROUP = B_PER_W // G
W = 1600
NWIN = B_PER_W // W
NKEY = NCHUNK * 16
KPAD = 112


@functools.partial(
    pl.kernel,
    out_type=jax.ShapeDtypeStruct((B_TOKENS, DIM), jnp.float32),
    mesh=plsc.VectorSubcoreMesh(core_axis_name="c", subcore_axis_name="s"),
    compiler_params=pltpu.CompilerParams(needs_layout_passes=False),
    scratch_types=[
        pltpu.VMEM((W,), jnp.int32),
        pltpu.VMEM((W,), jnp.int32),
        pltpu.VMEM((KPAD,), jnp.int32),
        pltpu.VMEM((KPAD,), jnp.int32),
        pltpu.VMEM((B_PER_W,), jnp.int32),
        pltpu.VMEM((4, G), jnp.int32),
        pltpu.VMEM((4, G), jnp.int32),
        pltpu.VMEM((G,), jnp.int32),
        pltpu.VMEM((4, G, DIM), jnp.float32),
        pltpu.VMEM_SHARED((SROWS, DIM), jnp.float32),
    ] + [pltpu.SemaphoreType.DMA] * 10,
)
def _gather_kernel(table_hbm, idx_hbm, out_hbm, winbuf0, winbuf1, cur_v,
                   start_v, pkb, posr, idxr, clb, rows_v, spm, *sems):
    winbufs = (winbuf0, winbuf1)
    winsem = sems[:2]
    gsem = sems[2:6]
    wsem = sems[6:10]
    sid = lax.axis_index("s")
    wid = sid * NC + lax.axis_index("c")
    base = wid * B_PER_W
    lane = lax.iota(jnp.int32, 16)
    ones = jnp.ones((16,), jnp.int32)
    zeros = jnp.zeros((16,), jnp.int32)

    def win_desc(w, b):
        return pltpu.make_async_copy(
            idx_hbm.at[pl.ds(base + w * W, W)], winbufs[b], winsem[b])

    def scan(per_vreg):
        win_desc(0, 0).start()
        win_desc(1, 1).start()
        for w in range(NWIN):
            b = w % 2
            win_desc(w, b).wait()

            def body(k, _):
                per_vreg(winbufs[b][pl.ds(k * 16, 16)], w * W + k * 16 + lane)
                return ()

            lax.fori_loop(0, W // 16, body, (), unroll=4)
            if w + 2 < NWIN:
                win_desc(w + 2, b).start()

    for k in range(KPAD // 16):
        cur_v[pl.ds(k * 16, 16)] = zeros

    def hist(v, _):
        key = ((v >> CSH) << 4) | lane
        plsc.addupdate_scatter(cur_v, [key], ones)

    scan(hist)

    total = jnp.int32(0)
    for k in range(KPAD // 16):
        v = cur_v[pl.ds(k * 16, 16)]
        ex = plsc.cumsum(v) - v + total
        start_v[pl.ds(k * 16, 16)] = ex
        cur_v[pl.ds(k * 16, 16)] = ex
        total = total + jnp.sum(v)

    def place(v, gpos):
        cid = v >> CSH
        key = (cid << 4) | lane
        off = plsc.load_gather(cur_v, [key])
        plsc.addupdate_scatter(cur_v, [key], ones)
        pk = (v & (SROWS - 1)) | (gpos << CSH) | (cid << (CSH + 15))
        plsc.store_scatter(pkb, [off], pk)

    scan(place)

    s_chunk = [start_v[pl.ds(16 * p, 16)][0] for p in range(NCHUNK)] + [jnp.int32(B_PER_W)]

    def gather_desc(b):
        return pltpu.make_async_copy(
            spm.at[idxr.at[b]], rows_v.at[b], gsem[b])

    def write_desc(b):
        return pltpu.make_async_copy(
            rows_v.at[b], out_hbm.at[posr.at[b]], wsem[b])

    def unpack(g, b):
        for j in range(G // 16):
            pk = pkb[pl.ds(g * G + j * 16, 16)]
            idxr[b, pl.ds(j * 16, 16)] = pk & (SROWS - 1)
            posr[b, pl.ds(j * 16, 16)] = base + ((pk >> CSH) & 0x7FFF)

    for p in range(NCHUNK):
        plsc.subcore_barrier()
        if p < NCHUNK - 1:
            pltpu.sync_copy(
                table_hbm.at[pl.ds(p * SROWS + sid * (SROWS // 16), SROWS // 16)],
                spm.at[pl.ds(sid * (SROWS // 16), SROWS // 16)])
        else:
            rem = VOCAB - (NCHUNK - 1) * SROWS
            per = 64
            pltpu.sync_copy(
                table_hbm.at[pl.ds((NCHUNK - 1) * SROWS + sid * per, per)],
                spm.at[pl.ds(sid * per, per)])
            @pl.when(sid == 0)
            def _():
                tail = rem - 16 * per
                pltpu.sync_copy(
                    table_hbm.at[pl.ds((NCHUNK - 1) * SROWS + 16 * per, tail)],
                    spm.at[pl.ds(16 * per, tail)])
        plsc.subcore_barrier()

        glo = s_chunk[p] >> 6
        ghi = (s_chunk[p + 1] >> 6) if p < NCHUNK - 1 else jnp.int32(NGROUP)
        n = ghi - glo

        for s in range(2):
            @pl.when(glo + s < ghi)
            def _(s=s):
                unpack(glo + s, s)
                gather_desc(s).start()

        def quad(h, _):
            for b in range(4):
                g = glo + 4 * h + b
                gp = g + 2
                bp = (b + 2) % 4
                wcond = (gp < ghi) & (h > 0) if b < 2 else (gp < ghi)

                @pl.when(wcond)
                def _(bp=bp):
                    write_desc(bp).wait()

                @pl.when(gp < ghi)
                def _(gp=gp, bp=bp):
                    unpack(gp, bp)
                    gather_desc(bp).start()

                @pl.when(g < ghi)
                def _(g=g, b=b):
                    gather_desc(b).wait()
                    write_desc(b).start()
            return ()

        lax.fori_loop(0, (n + 3) >> 2, quad, ())
        for b in range(4):
            @pl.when(n >= b + 1)
            def _(b=b):
                write_desc(b).wait()

    for p in range(1, NCHUNK):
        gb = s_chunk[p] >> 6

        @pl.when(gb < NGROUP)
        def _():
            for j in range(G // 16):
                pk = pkb[pl.ds(gb * G + j * 16, 16)]
                clb[pl.ds(j * 16, 16)] = (pk & (SROWS - 1)) | ((pk >> (CSH + 15)) << CSH)
                posr[0, pl.ds(j * 16, 16)] = base + ((pk >> CSH) & 0x7FFF)
            pltpu.make_async_copy(table_hbm.at[clb], rows_v.at[0], gsem[0]).start()
            pltpu.make_async_copy(table_hbm.at[clb], rows_v.at[0], gsem[0]).wait()
            write_desc(0).start()
            write_desc(0).wait()


def kernel(x, table):
    idx = x.reshape(-1).astype(jnp.int32)
    out = _gather_kernel(table, idx)
    return out.reshape(x.shape[0], x.shape[1], DIM)
